# Initial kernel scaffold; baseline (speedup 1.0000x reference)
import jax
import jax.numpy as jnp
from jax.experimental import pallas as pl


def kernel(prediction, zone):
    raise NotImplementedError("write your pallas kernel here")


# speedup vs baseline: 47.0507x; 47.0507x over previous
import jax
import jax.numpy as jnp
from jax.experimental import pallas as pl

CLASS_NUM = 80
CONF_THRE = 0.2
NMS_THRE = 0.45

N = 5000
NPAD = 5120
ROWS = 8
COLS = 640


def _dm_kernel(pred_ref, zone_ref,
               y1o, x1o, y2o, x2o, inzko, sco, clso, cyo, cxo, keepo):
    f32 = jnp.float32
    cx = pred_ref[0]
    cy = pred_ref[1]
    w = pred_ref[2]
    h = pred_ref[3]
    obj = pred_ref[4]
    x1 = cx - w / 2
    y1 = cy - h / 2
    x2 = cx + w / 2
    y2 = cy + h / 2
    area = jnp.clip(x2 - x1, 0.0) * jnp.clip(y2 - y1, 0.0)

    def cbody(k, carry):
        best, bk = carry
        v = pred_ref[5 + k]
        gt = v > best
        return jnp.where(gt, v, best), jnp.where(gt, k, bk)

    best0 = pred_ref[5]
    bk0 = jnp.zeros((ROWS, COLS), jnp.int32)
    class_conf, class_pred = jax.lax.fori_loop(1, CLASS_NUM, cbody, (best0, bk0))

    snms = obj * class_conf
    valid = snms >= CONF_THRE
    s0 = jnp.where(valid, snms, f32(-1.0))

    iota = (jax.lax.broadcasted_iota(jnp.int32, (ROWS, COLS), 0) * COLS
            + jax.lax.broadcasted_iota(jnp.int32, (ROWS, COLS), 1))
    keep0 = jnp.zeros((ROWS, COLS), f32)
    m0 = jnp.max(s0)

    def nms_cond(carry):
        _, _, m = carry
        return m >= CONF_THRE

    def nms_body(carry):
        s, keepf, m = carry
        i = jnp.min(jnp.where(s == m, iota, jnp.int32(NPAD)))
        sel = iota == i
        ninf = f32(-jnp.inf)
        x1s = jnp.max(jnp.where(sel, x1, ninf))
        y1s = jnp.max(jnp.where(sel, y1, ninf))
        x2s = jnp.max(jnp.where(sel, x2, ninf))
        y2s = jnp.max(jnp.where(sel, y2, ninf))
        areas = jnp.clip(x2s - x1s, 0.0) * jnp.clip(y2s - y1s, 0.0)
        ltx = jnp.maximum(x1s, x1)
        lty = jnp.maximum(y1s, y1)
        rbx = jnp.minimum(x2s, x2)
        rby = jnp.minimum(y2s, y2)
        iw = jnp.clip(rbx - ltx, 0.0)
        ih = jnp.clip(rby - lty, 0.0)
        inter = iw * ih
        union = areas + area - inter
        iou = inter / jnp.maximum(union, f32(1e-9))
        kill = (iou > NMS_THRE) | sel
        s2 = jnp.where(kill, f32(-1.0), s)
        keep2 = jnp.where(sel, f32(1.0), keepf)
        return s2, keep2, jnp.max(s2)

    _, keepf, _ = jax.lax.while_loop(nms_cond, nms_body, (s0, keep0, m0))
    keepb = keepf > 0.0

    px = (x1 + x2) / 2
    py = (y1 + y2) / 2

    parity = jnp.zeros((ROWS, COLS), jnp.bool_)
    for k in range(8):
        xi = zone_ref[k, 0]
        yi = zone_ref[k, 1]
        xj = zone_ref[(k - 1) % 8, 0]
        yj = zone_ref[(k - 1) % 8, 1]
        gyi = yi > py
        gyj = yj > py
        gx = (xj - xi) * (py - yi) / (yj - yi) + xi
        parity = parity ^ ((gyi != gyj) & (gx > px))

    mk = keepf
    y1o[...] = y1 * mk
    x1o[...] = x1 * mk
    y2o[...] = y2 * mk
    x2o[...] = x2 * mk
    inzko[...] = (parity & keepb).astype(jnp.int32)
    sco[...] = jnp.maximum(obj, class_conf) * mk
    clso[...] = jnp.where(keepb, class_pred, -1)
    cyo[...] = py * mk
    cxo[...] = px * mk
    keepo[...] = keepb.astype(jnp.int32)


def kernel(prediction, zone):
    p = prediction[0]
    pT = jnp.pad(jnp.transpose(p), ((0, 0), (0, NPAD - N)))
    pp = pT.reshape(85, ROWS, COLS)

    f32 = jnp.float32
    outs = pl.pallas_call(
        _dm_kernel,
        out_shape=[
            jax.ShapeDtypeStruct((ROWS, COLS), f32),
            jax.ShapeDtypeStruct((ROWS, COLS), f32),
            jax.ShapeDtypeStruct((ROWS, COLS), f32),
            jax.ShapeDtypeStruct((ROWS, COLS), f32),
            jax.ShapeDtypeStruct((ROWS, COLS), jnp.int32),
            jax.ShapeDtypeStruct((ROWS, COLS), f32),
            jax.ShapeDtypeStruct((ROWS, COLS), jnp.int32),
            jax.ShapeDtypeStruct((ROWS, COLS), f32),
            jax.ShapeDtypeStruct((ROWS, COLS), f32),
            jax.ShapeDtypeStruct((ROWS, COLS), jnp.int32),
        ],
    )(pp, zone)

    y1m, x1m, y2m, x2m, inzk, sc, cls_o, cym, cxm, keep = [
        o.reshape(NPAD)[:N] for o in outs
    ]
    boxes_yxyx = jnp.stack([y1m, x1m, y2m, x2m], axis=1)
    centers_yx = jnp.stack([cym, cxm], axis=1)
    return (boxes_yxyx,
            inzk.astype(jnp.bool_),
            sc,
            cls_o,
            centers_yx,
            keep.astype(jnp.bool_))


# speedup vs baseline: 61.4476x; 1.3060x over previous
import jax
import jax.numpy as jnp
from jax.experimental import pallas as pl
from jax.experimental.pallas import tpu as pltpu

CLASS_NUM = 80
CONF_THRE = 0.2
NMS_THRE = 0.45

N = 5000
NPAD = 5120
ROWS = 8
COLS = 640


def _dm_kernel(pred_ref, pred4_ref, zone_ref,
               y1o, x1o, y2o, x2o, inzko, sco, clso, cyo, cxo, keepo,
               x1r, y1r, x2r, y2r, arear, iotar):
    f32 = jnp.float32
    cx = pred_ref[0]
    cy = pred_ref[1]
    w = pred_ref[2]
    h = pred_ref[3]
    obj = pred_ref[4]
    x1r[...] = cx - w / 2
    y1r[...] = cy - h / 2
    x2r[...] = cx + w / 2
    y2r[...] = cy + h / 2
    arear[...] = (jnp.clip(x2r[...] - x1r[...], 0.0)
                  * jnp.clip(y2r[...] - y1r[...], 0.0))
    iotar[...] = (jax.lax.broadcasted_iota(jnp.int32, (ROWS, COLS), 0) * COLS
                  + jax.lax.broadcasted_iota(jnp.int32, (ROWS, COLS), 1))

    def cbody(k, carry):
        best, bk = carry
        v = pred_ref[5 + k]
        gt = v > best
        return jnp.where(gt, v, best), jnp.where(gt, k, bk)

    best0 = pred_ref[5]
    bk0 = jnp.zeros((ROWS, COLS), jnp.int32)
    class_conf, class_pred = jax.lax.fori_loop(1, CLASS_NUM, cbody, (best0, bk0))

    snms = obj * class_conf
    valid = snms >= CONF_THRE
    s0 = jnp.where(valid, snms, f32(-1.0))
    m0 = jnp.max(s0)

    def nms_cond(carry):
        _, m = carry
        return m >= CONF_THRE

    def nms_body(carry):
        s, m = carry
        iota = iotar[...]
        i = jnp.min(jnp.where(s == m, iota, jnp.int32(NPAD)))
        cxs = pred4_ref[0, i]
        cys = pred4_ref[1, i]
        ws = pred4_ref[2, i]
        hs = pred4_ref[3, i]
        x1s = cxs - ws / 2
        y1s = cys - hs / 2
        x2s = cxs + ws / 2
        y2s = cys + hs / 2
        areas = jnp.clip(x2s - x1s, 0.0) * jnp.clip(y2s - y1s, 0.0)
        ltx = jnp.maximum(x1s, x1r[...])
        lty = jnp.maximum(y1s, y1r[...])
        rbx = jnp.minimum(x2s, x2r[...])
        rby = jnp.minimum(y2s, y2r[...])
        iw = jnp.clip(rbx - ltx, 0.0)
        ih = jnp.clip(rby - lty, 0.0)
        inter = iw * ih
        union = areas + arear[...] - inter
        iou = inter / jnp.maximum(union, f32(1e-9))
        sel = iota == i
        s2 = jnp.where(sel, f32(-2.0), jnp.where(iou > NMS_THRE, f32(-1.0), s))
        return s2, jnp.max(s2)

    sf, _ = jax.lax.while_loop(nms_cond, nms_body, (s0, m0))
    keepb = sf == f32(-2.0)
    mk = jnp.where(keepb, f32(1.0), f32(0.0))

    x1 = x1r[...]
    y1 = y1r[...]
    x2 = x2r[...]
    y2 = y2r[...]
    px = (x1 + x2) / 2
    py = (y1 + y2) / 2

    parity = jnp.zeros((ROWS, COLS), jnp.bool_)
    for k in range(8):
        xi = zone_ref[k, 0]
        yi = zone_ref[k, 1]
        xj = zone_ref[(k - 1) % 8, 0]
        yj = zone_ref[(k - 1) % 8, 1]
        gyi = yi > py
        gyj = yj > py
        gx = (xj - xi) * (py - yi) / (yj - yi) + xi
        parity = parity ^ ((gyi != gyj) & (gx > px))

    y1o[...] = y1 * mk
    x1o[...] = x1 * mk
    y2o[...] = y2 * mk
    x2o[...] = x2 * mk
    inzko[...] = (parity & keepb).astype(jnp.int32)
    sco[...] = jnp.maximum(obj, class_conf) * mk
    clso[...] = jnp.where(keepb, class_pred, -1)
    cyo[...] = py * mk
    cxo[...] = px * mk
    keepo[...] = keepb.astype(jnp.int32)


def kernel(prediction, zone):
    p = prediction[0]
    pT = jnp.pad(jnp.transpose(p), ((0, 0), (0, NPAD - N)))
    pp = pT.reshape(85, ROWS, COLS)
    pred4 = pT[:4]

    f32 = jnp.float32
    outs = pl.pallas_call(
        _dm_kernel,
        in_specs=[
            pl.BlockSpec(memory_space=pltpu.VMEM),
            pl.BlockSpec(memory_space=pltpu.SMEM),
            pl.BlockSpec(memory_space=pltpu.SMEM),
        ],
        out_shape=[
            jax.ShapeDtypeStruct((ROWS, COLS), f32),
            jax.ShapeDtypeStruct((ROWS, COLS), f32),
            jax.ShapeDtypeStruct((ROWS, COLS), f32),
            jax.ShapeDtypeStruct((ROWS, COLS), f32),
            jax.ShapeDtypeStruct((ROWS, COLS), jnp.int32),
            jax.ShapeDtypeStruct((ROWS, COLS), f32),
            jax.ShapeDtypeStruct((ROWS, COLS), jnp.int32),
            jax.ShapeDtypeStruct((ROWS, COLS), f32),
            jax.ShapeDtypeStruct((ROWS, COLS), f32),
            jax.ShapeDtypeStruct((ROWS, COLS), jnp.int32),
        ],
        scratch_shapes=[
            pltpu.VMEM((ROWS, COLS), f32),
            pltpu.VMEM((ROWS, COLS), f32),
            pltpu.VMEM((ROWS, COLS), f32),
            pltpu.VMEM((ROWS, COLS), f32),
            pltpu.VMEM((ROWS, COLS), f32),
            pltpu.VMEM((ROWS, COLS), jnp.int32),
        ],
    )(pp, pred4, zone)

    y1m, x1m, y2m, x2m, inzk, sc, cls_o, cym, cxm, keep = [
        o.reshape(NPAD)[:N] for o in outs
    ]
    boxes_yxyx = jnp.stack([y1m, x1m, y2m, x2m], axis=1)
    centers_yx = jnp.stack([cym, cxm], axis=1)
    return (boxes_yxyx,
            inzk.astype(jnp.bool_),
            sc,
            cls_o,
            centers_yx,
            keep.astype(jnp.bool_))
